# baseline (speedup 1.0000x reference)
import jax
import jax.numpy as jnp
from jax import lax
from jax.experimental import pallas as pl
from jax.experimental.pallas import tpu as pltpu
from jax.experimental.pallas import tpu_sc as plsc

BATCH = 16384
EMBED_DIM = 32
NUM_CORES = 2
LANES = 16
NUM_WORKERS = NUM_CORES * 16
B_PER_W = BATCH // NUM_WORKERS
GROUPS = B_PER_W // LANES


CHUNK = 256
N_CHUNKS = B_PER_W // CHUNK
CGROUPS = CHUNK // LANES


def _dot_kernel(user_hbm, mission_hbm, uemb_hbm, memb_hbm, out_hbm,
                uidx_v, midx_v, uq_v, mq_v, urows_v, mrows_v, out_v,
                sem_u, sem_m):
    wid = lax.axis_index("s") * NUM_CORES + lax.axis_index("c")
    base = wid * B_PER_W

    pltpu.sync_copy(user_hbm.at[pl.ds(base, B_PER_W)], uidx_v)
    pltpu.sync_copy(mission_hbm.at[pl.ds(base, B_PER_W)], midx_v)

    lane_iota = lax.iota(jnp.int32, LANES)

    def chunk_body(c, carry):
        coff = c * CHUNK

        def quot_body(g, carry2):
            qoff = g * LANES
            uq_v[pl.ds(qoff, LANES)] = (
                uidx_v[pl.ds(coff + qoff, LANES)] >> 2)
            mq_v[pl.ds(qoff, LANES)] = (
                midx_v[pl.ds(coff + qoff, LANES)] >> 2)
            return carry2

        lax.fori_loop(0, CGROUPS, quot_body, 0)

        cp_u = pltpu.async_copy(uemb_hbm.at[uq_v], urows_v, sem_u)
        cp_m = pltpu.async_copy(memb_hbm.at[mq_v], mrows_v, sem_m)
        cp_u.wait()
        cp_m.wait()

        def group_body(g, carry2):
            off = g * LANES
            rows = off + lane_iota
            ucol0 = (uidx_v[pl.ds(coff + off, LANES)] & 3) << 5
            mcol0 = (midx_v[pl.ds(coff + off, LANES)] & 3) << 5
            acc = jnp.zeros((LANES,), jnp.float32)
            for d in range(EMBED_DIM):
                uv = plsc.load_gather(urows_v, [rows, ucol0 + d])
                mv = plsc.load_gather(mrows_v, [rows, mcol0 + d])
                acc = acc + uv * mv
            out_v[pl.ds(coff + off, LANES)] = acc
            return carry2

        lax.fori_loop(0, CGROUPS, group_body, 0)
        return carry

    lax.fori_loop(0, N_CHUNKS, chunk_body, 0)

    pltpu.sync_copy(out_v, out_hbm.at[pl.ds(base, B_PER_W)])


def _bias_kernel(user_hbm, mission_hbm, ubias_hbm, mbias_hbm, bias_hbm,
                 dot_hbm, out_hbm,
                 uidx_v, midx_v, ub_v, mb_v, bias_v, dot_v, out_v,
                 sem_ub, sem_mb, sem_d):
    wid = lax.axis_index("s") * NUM_CORES + lax.axis_index("c")
    base = wid * B_PER_W

    bias_v[...] = jnp.zeros((LANES,), jnp.float32)
    zidx = uidx_v
    zidx[pl.ds(0, LANES)] = jnp.zeros((LANES,), jnp.int32)
    pltpu.sync_copy(bias_hbm.at[zidx.at[pl.ds(0, LANES)]], bias_v)
    bias_vec = bias_v[...]

    pltpu.sync_copy(user_hbm.at[pl.ds(base, B_PER_W)], uidx_v)
    pltpu.sync_copy(mission_hbm.at[pl.ds(base, B_PER_W)], midx_v)

    cp_d = pltpu.async_copy(dot_hbm.at[pl.ds(base, B_PER_W)], dot_v, sem_d)
    cp_ub = pltpu.async_copy(ubias_hbm.at[uidx_v], ub_v, sem_ub)
    cp_mb = pltpu.async_copy(mbias_hbm.at[midx_v], mb_v, sem_mb)
    cp_d.wait()
    cp_ub.wait()
    cp_mb.wait()

    def group_body(g, carry):
        off = g * LANES
        out_v[pl.ds(off, LANES)] = (dot_v[pl.ds(off, LANES)]
                                    + ub_v[pl.ds(off, LANES)]
                                    + mb_v[pl.ds(off, LANES)]
                                    + bias_vec)
        return carry

    lax.fori_loop(0, GROUPS, group_body, 0)

    pltpu.sync_copy(out_v, out_hbm.at[pl.ds(base, B_PER_W)])


@jax.jit
def _run(user, mission, uemb, memb, ubias, mbias, bias):
    mesh = plsc.VectorSubcoreMesh(core_axis_name="c", subcore_axis_name="s")

    dot_fn = pl.kernel(
        _dot_kernel,
        out_type=jax.ShapeDtypeStruct((BATCH,), jnp.float32),
        mesh=mesh,
        compiler_params=pltpu.CompilerParams(needs_layout_passes=False,
                                             use_tc_tiling_on_sc=True),
        scratch_types=[
            pltpu.VMEM((B_PER_W,), jnp.int32),
            pltpu.VMEM((B_PER_W,), jnp.int32),
            pltpu.VMEM((CHUNK,), jnp.int32),
            pltpu.VMEM((CHUNK,), jnp.int32),
            pltpu.VMEM((CHUNK, 128), jnp.float32),
            pltpu.VMEM((CHUNK, 128), jnp.float32),
            pltpu.VMEM((B_PER_W,), jnp.float32),
            pltpu.SemaphoreType.DMA,
            pltpu.SemaphoreType.DMA,
        ],
    )
    dot = dot_fn(user, mission, uemb.reshape(-1, 128), memb.reshape(-1, 128))

    bias_fn = pl.kernel(
        _bias_kernel,
        out_type=jax.ShapeDtypeStruct((BATCH,), jnp.float32),
        mesh=mesh,
        compiler_params=pltpu.CompilerParams(needs_layout_passes=False,
                                             use_tc_tiling_on_sc=False),
        scratch_types=[
            pltpu.VMEM((B_PER_W,), jnp.int32),
            pltpu.VMEM((B_PER_W,), jnp.int32),
            pltpu.VMEM((B_PER_W,), jnp.float32),
            pltpu.VMEM((B_PER_W,), jnp.float32),
            pltpu.VMEM((LANES,), jnp.float32),
            pltpu.VMEM((B_PER_W,), jnp.float32),
            pltpu.VMEM((B_PER_W,), jnp.float32),
            pltpu.SemaphoreType.DMA,
            pltpu.SemaphoreType.DMA,
            pltpu.SemaphoreType.DMA,
        ],
    )
    return bias_fn(user, mission, ubias, mbias, bias, dot)


def kernel(user, mission, user_embedding, mission_embedding, user_bias,
           mission_bias, bias):
    user = user.astype(jnp.int32)
    mission = mission.astype(jnp.int32)
    return _run(user, mission, user_embedding, mission_embedding,
                user_bias.reshape(-1), mission_bias.reshape(-1),
                bias.reshape(-1))
